# Initial kernel scaffold; baseline (speedup 1.0000x reference)
import jax
import jax.numpy as jnp
from jax.experimental import pallas as pl


def kernel(x, y):
    raise NotImplementedError("write your pallas kernel here")


# speedup vs baseline: 1.0059x; 1.0059x over previous
import functools

import jax
import jax.numpy as jnp
from jax.experimental import pallas as pl
from jax.experimental.pallas import tpu as pltpu

_PATCH = 3
_PAD = _PATCH // 2
_NUM_PAIRS = 4


def _loss_kernel(mask_ref, x_ref, y_ref, out_ref, *, HWp, W, R, CSUB):
    masks = [mask_ref[pl.ds(i, 1), :] for i in range(_NUM_PAIRS)]

    def chunk_partial(xc, yc):
        xw = pltpu.roll(xc, shift=HWp - W, axis=1)
        yw = pltpu.roll(yc, shift=HWp - W, axis=1)
        nbrs = (
            (pltpu.roll(xc, HWp - 1, axis=1), pltpu.roll(yc, HWp - 1, axis=1),
             masks[0]),
            (pltpu.roll(xw, 1, axis=1), pltpu.roll(yw, 1, axis=1),
             masks[1]),
            (xw, yw, masks[2]),
            (pltpu.roll(xw, HWp - 1, axis=1), pltpu.roll(yw, HWp - 1, axis=1),
             masks[3]),
        )
        acc = jnp.zeros(xc.shape, jnp.float32)
        for px, py, mm in nbrs:
            dx = px - xc
            dy = py - yc
            nx = dx * jax.lax.rsqrt(dx * dx + 0.81)
            ny = dy * jax.lax.rsqrt(dy * dy + 0.81)
            d = nx - ny
            r = pl.reciprocal(d * d + 0.1, approx=True)
            acc = acc + mm * r
        return jnp.sum(acc)

    n_chunks = R // CSUB

    def body(ci, carry):
        c0 = pl.multiple_of(ci * CSUB, CSUB)
        return carry + chunk_partial(x_ref[pl.ds(c0, CSUB), :],
                                     y_ref[pl.ds(c0, CSUB), :])

    total = jax.lax.fori_loop(0, n_chunks, body, jnp.zeros((), jnp.float32))
    out_ref[...] = total.reshape(1, 1, 1)


@jax.jit
def _geometry_loss(x, y):
    B, C, H, W = x.shape
    HW = H * W
    HWp = max(128, ((HW + 127) // 128) * 128)
    BC = B * C

    xf = x.astype(jnp.float32).reshape(BC, HW)
    yf = y.astype(jnp.float32).reshape(BC, HW)
    if HWp != HW:
        pad = ((0, 0), (0, HWp - HW))
        xf = jnp.pad(xf, pad)
        yf = jnp.pad(yf, pad)

    R = BC
    for cand in (BC // 8, BC // 4, BC // 2, BC):
        if cand and BC % cand == 0 and cand % 8 == 0:
            R = cand
            break
    n_cells = BC // R
    CSUB = 8 if R % 8 == 0 else R

    row = jnp.arange(H)[:, None]
    col = jnp.arange(W)[None, :]
    interior2d = ((row >= _PAD) & (row < H - _PAD) &
                  (col >= _PAD) & (col < W - _PAD)).astype(jnp.float32)
    interior = jnp.pad(interior2d.reshape(HW), (0, HWp - HW))
    pair_offsets = (1, W - 1, W, W + 1)
    masks = jnp.stack(
        [interior + jnp.roll(interior, -s) for s in pair_offsets], axis=0)

    kernel_body = functools.partial(_loss_kernel, HWp=HWp, W=W, R=R, CSUB=CSUB)

    partials = pl.pallas_call(
        kernel_body,
        out_shape=jax.ShapeDtypeStruct((n_cells, 1, 1), jnp.float32),
        grid=(n_cells,),
        in_specs=[
            pl.BlockSpec((_NUM_PAIRS, HWp), lambda g: (0, 0)),
            pl.BlockSpec((R, HWp), lambda g: (g, 0)),
            pl.BlockSpec((R, HWp), lambda g: (g, 0)),
        ],
        out_specs=pl.BlockSpec((1, 1, 1), lambda g: (g, 0, 0)),
        compiler_params=pltpu.CompilerParams(
            dimension_semantics=("parallel",),
            vmem_limit_bytes=int(48 << 20)),
    )(masks, xf, yf)

    mask_total = jnp.sum(masks) * BC
    grand = mask_total - 0.1 * jnp.sum(partials)
    return grand / (C * _PATCH * _PATCH * B * H * W)


def kernel(x, y):
    return _geometry_loss(x, y)


# speedup vs baseline: 1.0421x; 1.0359x over previous
import functools

import numpy as np

import jax
import jax.numpy as jnp
from jax.experimental import pallas as pl
from jax.experimental.pallas import tpu as pltpu

_PATCH = 3
_PAD = _PATCH // 2
_NUM_PAIRS = 4


def _loss_kernel(mask_ref, x_ref, y_ref, out_ref, *, HWp, W, R, CSUB,
                 n_cells, mask_const, inv_norm):
    g = pl.program_id(0)

    @pl.when(g == 0)
    def _():
        out_ref[...] = jnp.zeros_like(out_ref)

    masks = [mask_ref[pl.ds(i, 1), :] for i in range(_NUM_PAIRS)]

    def chunk_partial(xc, yc):
        xp = pltpu.roll(xc, HWp - 1, axis=1)
        xm = pltpu.roll(xc, 1, axis=1)
        yp = pltpu.roll(yc, HWp - 1, axis=1)
        ym = pltpu.roll(yc, 1, axis=1)
        free = lambda a: pltpu.roll(a, HWp - W, axis=1)
        nbrs = (
            (xp, yp, masks[0]),
            (free(xm), free(ym), masks[1]),
            (free(xc), free(yc), masks[2]),
            (free(xp), free(yp), masks[3]),
        )
        acc = jnp.zeros(xc.shape, jnp.float32)
        for px, py, mm in nbrs:
            dx = px - xc
            dy = py - yc
            nx = dx * jax.lax.rsqrt(dx * dx + 0.81)
            ny = dy * jax.lax.rsqrt(dy * dy + 0.81)
            d = nx - ny
            r = pl.reciprocal(d * d + 0.1, approx=True)
            acc = acc + mm * r
        return jnp.sum(acc)

    n_chunks = R // CSUB

    def body(ci, carry):
        c0 = pl.multiple_of(ci * CSUB, CSUB)
        return carry + chunk_partial(x_ref[pl.ds(c0, CSUB), :],
                                     y_ref[pl.ds(c0, CSUB), :])

    cell = jax.lax.fori_loop(0, n_chunks, body, jnp.zeros((), jnp.float32))
    out_ref[...] += cell.reshape(1, 1)

    @pl.when(g == n_cells - 1)
    def _():
        total = out_ref[0, 0]
        out_ref[...] = ((mask_const - 0.1 * total) * inv_norm).reshape(1, 1)


@jax.jit
def _geometry_loss(x, y):
    B, C, H, W = x.shape
    HW = H * W
    HWp = max(128, ((HW + 127) // 128) * 128)
    BC = B * C

    xf = x.astype(jnp.float32).reshape(BC, HW)
    yf = y.astype(jnp.float32).reshape(BC, HW)
    if HWp != HW:
        pad = ((0, 0), (0, HWp - HW))
        xf = jnp.pad(xf, pad)
        yf = jnp.pad(yf, pad)

    R = BC
    for cand in (BC // 8, BC // 4, BC // 2, BC):
        if cand and BC % cand == 0 and cand % 8 == 0:
            R = cand
            break
    n_cells = BC // R
    CSUB = 8 if R % 8 == 0 else R

    row = np.arange(H)[:, None]
    col = np.arange(W)[None, :]
    interior2d = ((row >= _PAD) & (row < H - _PAD) &
                  (col >= _PAD) & (col < W - _PAD)).astype(np.float32)
    interior = np.pad(interior2d.reshape(HW), (0, HWp - HW))
    pair_offsets = (1, W - 1, W, W + 1)
    np_masks = np.stack(
        [interior + np.roll(interior, -s) for s in pair_offsets], axis=0)
    masks = jnp.asarray(np_masks)

    mask_const = float(np_masks.sum()) * BC
    inv_norm = 1.0 / (C * _PATCH * _PATCH * B * H * W)

    kernel_body = functools.partial(
        _loss_kernel, HWp=HWp, W=W, R=R, CSUB=CSUB, n_cells=n_cells,
        mask_const=mask_const, inv_norm=inv_norm)

    out = pl.pallas_call(
        kernel_body,
        out_shape=jax.ShapeDtypeStruct((1, 1), jnp.float32),
        grid=(n_cells,),
        in_specs=[
            pl.BlockSpec((_NUM_PAIRS, HWp), lambda g: (0, 0)),
            pl.BlockSpec((R, HWp), lambda g: (g, 0)),
            pl.BlockSpec((R, HWp), lambda g: (g, 0)),
        ],
        out_specs=pl.BlockSpec((1, 1), lambda g: (0, 0)),
        compiler_params=pltpu.CompilerParams(
            dimension_semantics=("arbitrary",),
            vmem_limit_bytes=int(40 << 20)),
    )(masks, xf, yf)

    return out[0, 0]


def kernel(x, y):
    return _geometry_loss(x, y)


# speedup vs baseline: 1.8571x; 1.7821x over previous
import functools

import numpy as np

import jax
import jax.numpy as jnp
from jax.experimental import pallas as pl
from jax.experimental.pallas import tpu as pltpu

_PATCH = 3
_PAD = _PATCH // 2
_NUM_PAIRS = 4


def _loss_kernel(mask_ref, x_ref, y_ref, out_ref, *, H, W, TC, CSUB,
                 n_steps, mask_const, inv_norm):
    step = pl.program_id(0) * pl.num_programs(1) + pl.program_id(1)

    @pl.when(step == 0)
    def _():
        out_ref[...] = jnp.zeros_like(out_ref)

    masks = [mask_ref[pl.ds(i, 1), :, :] for i in range(_NUM_PAIRS)]

    def chunk_partial(xc, yc):
        xd = pltpu.roll(xc, H - 1, axis=1)
        yd = pltpu.roll(yc, H - 1, axis=1)
        rp = lambda a: pltpu.roll(a, W - 1, axis=2)
        rm = lambda a: pltpu.roll(a, 1, axis=2)
        nbrs = (
            (rp(xc), rp(yc), masks[0]),
            (rm(xd), rm(yd), masks[1]),
            (xd, yd, masks[2]),
            (rp(xd), rp(yd), masks[3]),
        )
        acc = jnp.zeros(xc.shape, jnp.float32)
        for px, py, mm in nbrs:
            dx = px - xc
            dy = py - yc
            nx = dx * jax.lax.rsqrt(dx * dx + 0.81)
            ny = dy * jax.lax.rsqrt(dy * dy + 0.81)
            d = nx - ny
            r = pl.reciprocal(d * d + 0.1, approx=True)
            acc = acc + mm * r
        return jnp.sum(acc)

    n_chunks = TC // CSUB

    def body(ci, carry):
        c0 = pl.multiple_of(ci * CSUB, CSUB)
        return carry + chunk_partial(x_ref[0, pl.ds(c0, CSUB), :, :],
                                     y_ref[0, pl.ds(c0, CSUB), :, :])

    cell = jax.lax.fori_loop(0, n_chunks, body, jnp.zeros((), jnp.float32))
    out_ref[...] += cell.reshape(1, 1)

    @pl.when(step == n_steps - 1)
    def _():
        total = out_ref[0, 0]
        out_ref[...] = ((mask_const - 0.1 * total) * inv_norm).reshape(1, 1)


@jax.jit
def _geometry_loss(x, y):
    B, C, H, W = x.shape
    HW = H * W

    xf = x.astype(jnp.float32)
    yf = y.astype(jnp.float32)

    TC = C
    for cand in (32, 64, C):
        if C % cand == 0:
            TC = cand
            break
    n_ct = C // TC
    n_steps = B * n_ct
    CSUB = 8 if TC % 8 == 0 else TC

    row = np.arange(H)[:, None]
    col = np.arange(W)[None, :]
    interior2d = ((row >= _PAD) & (row < H - _PAD) &
                  (col >= _PAD) & (col < W - _PAD)).astype(np.float32)
    interior = interior2d.reshape(HW)
    pair_offsets = (1, W - 1, W, W + 1)
    np_masks = np.stack(
        [(interior + np.roll(interior, -s)).reshape(H, W)
         for s in pair_offsets], axis=0)
    masks = jnp.asarray(np_masks)

    mask_const = float(np_masks.sum()) * B * C
    inv_norm = 1.0 / (C * _PATCH * _PATCH * B * H * W)

    kernel_body = functools.partial(
        _loss_kernel, H=H, W=W, TC=TC, CSUB=CSUB, n_steps=n_steps,
        mask_const=mask_const, inv_norm=inv_norm)

    out = pl.pallas_call(
        kernel_body,
        out_shape=jax.ShapeDtypeStruct((1, 1), jnp.float32),
        grid=(B, n_ct),
        in_specs=[
            pl.BlockSpec((_NUM_PAIRS, H, W), lambda b, c: (0, 0, 0)),
            pl.BlockSpec((1, TC, H, W), lambda b, c: (b, c, 0, 0)),
            pl.BlockSpec((1, TC, H, W), lambda b, c: (b, c, 0, 0)),
        ],
        out_specs=pl.BlockSpec((1, 1), lambda b, c: (0, 0)),
        compiler_params=pltpu.CompilerParams(
            dimension_semantics=("arbitrary", "arbitrary"),
            vmem_limit_bytes=int(40 << 20)),
    )(masks, xf, yf)

    return out[0, 0]


def kernel(x, y):
    return _geometry_loss(x, y)
